# baseline (speedup 1.0000x reference)
import functools

import jax
import jax.numpy as jnp
from jax import lax
from jax.experimental import pallas as pl
from jax.experimental.pallas import tpu as pltpu
from jax.experimental.pallas import tpu_sc as plsc

N_SAMPLES = 3145728
N_CH = 8
N_RAYS = 65536
NC = 2
NS = 16
NW = NC * NS
CHUNK = N_SAMPLES // NW
BLK = 2048
NBLK = CHUNK // BLK
GRP = BLK // 16
MS_ROWS = BLK // 128 * N_CH
ROWS_PER_SUB = N_RAYS // NS
NBUF = 2


def _sc_segment_sum(ms_lin, w, ridx):
  mesh = plsc.VectorSubcoreMesh(core_axis_name="c", subcore_axis_name="s")

  @functools.partial(
      pl.kernel,
      out_type=jax.ShapeDtypeStruct((NC, N_RAYS, N_CH), jnp.float32),
      mesh=mesh,
      scratch_types=dict(
          acc=pltpu.VMEM_SHARED((N_RAYS, N_CH), jnp.float32),
          ms_v=tuple(pltpu.VMEM((MS_ROWS, 128), jnp.float32)
                     for _ in range(NBUF)),
          w_v=tuple(pltpu.VMEM((BLK,), jnp.float32) for _ in range(NBUF)),
          idx_v=tuple(pltpu.VMEM((BLK,), jnp.int32) for _ in range(NBUF)),
          vals_v=tuple(pltpu.VMEM((BLK, N_CH), jnp.float32)
                       for _ in range(NBUF)),
          in_sems=tuple(pltpu.SemaphoreType.DMA((3,)) for _ in range(NBUF)),
          sc_sems=tuple(pltpu.SemaphoreType.DMA for _ in range(NBUF)),
      ),
      compiler_params=pltpu.CompilerParams(use_tc_tiling_on_sc=False,
                                           needs_layout_passes=False),
  )
  def seg_sum(ms_hbm, w_hbm, idx_hbm, out_hbm, *, acc, ms_v, w_v, idx_v,
              vals_v, in_sems, sc_sems):
    cid = lax.axis_index("c")
    sid = lax.axis_index("s")
    wid = cid * NS + sid

    iota = lax.iota(jnp.int32, 16)
    hi = iota >> 3
    lo = iota & 7
    zeros16 = jnp.zeros((16,), jnp.float32)

    def zero_body(i, _):
      plsc.store_scatter(vals_v[0], [2 * i + hi, lo], zeros16)
      return 0

    lax.fori_loop(0, BLK // 2, zero_body, 0)
    for rep in range(ROWS_PER_SUB // BLK):
      row0 = sid * ROWS_PER_SUB + rep * BLK
      pltpu.sync_copy(vals_v[0], acc.at[pl.ds(row0, BLK)])
    plsc.subcore_barrier()

    def start_in(b, k):
      s0 = pl.multiple_of(wid * CHUNK + b * BLK, BLK)
      r0 = pl.multiple_of(s0 // 16, MS_ROWS)
      pltpu.async_copy(ms_hbm.at[pl.ds(r0, MS_ROWS)], ms_v[k],
                       in_sems[k].at[0])
      pltpu.async_copy(w_hbm.at[pl.ds(s0, BLK)], w_v[k], in_sems[k].at[1])
      pltpu.async_copy(idx_hbm.at[pl.ds(s0, BLK)], idx_v[k],
                       in_sems[k].at[2])

    def wait_in(b, k):
      s0 = pl.multiple_of(wid * CHUNK + b * BLK, BLK)
      r0 = pl.multiple_of(s0 // 16, MS_ROWS)
      pltpu.make_async_copy(ms_hbm.at[pl.ds(r0, MS_ROWS)], ms_v[k],
                            in_sems[k].at[0]).wait()
      pltpu.make_async_copy(w_hbm.at[pl.ds(s0, BLK)], w_v[k],
                            in_sems[k].at[1]).wait()
      pltpu.make_async_copy(idx_hbm.at[pl.ds(s0, BLK)], idx_v[k],
                            in_sems[k].at[2]).wait()

    def wait_scatter(k):
      pltpu.make_async_copy(vals_v[k], acc.at[idx_v[k]], sc_sems[k]).wait()

    def compute_block(b, k):
      def store_group(samp, prods):
        for c in range(N_CH):
          plsc.store_scatter(vals_v[k],
                             [samp, jnp.full((16,), c, jnp.int32)],
                             prods[c])

      def group_body(g, carry):
        prev_samp, prev_prods = carry
        col = pl.multiple_of((g % (128 // 16)) * 16, 16)
        rowb = (g // (128 // 16)) * N_CH
        samp = 16 * g + iota
        w16 = w_v[k][pl.ds(pl.multiple_of(16 * g, 16), 16)]
        loads = [ms_v[k][rowb + c, pl.ds(col, 16)] for c in range(N_CH)]
        store_group(prev_samp, prev_prods)
        prods = tuple(m * w16 for m in loads)
        return samp, prods

      carry0 = (iota, tuple(zeros16 for _ in range(N_CH)))
      last = lax.fori_loop(0, GRP, group_body, carry0, unroll=2)
      store_group(*last)

    start_in(0, 0)

    def super_body(sb, _):
      for k in range(NBUF):
        b = sb * NBUF + k
        nk = (k + 1) % NBUF
        @pl.when(b >= 1)
        def _():
          wait_scatter(nk)

        @pl.when(b + 1 < NBLK)
        def _():
          start_in(b + 1, nk)

        wait_in(b, k)
        compute_block(b, k)
        pltpu.async_copy(vals_v[k], acc.at[idx_v[k]], sc_sems[k], add=True)
      return 0

    lax.fori_loop(0, NBLK // NBUF, super_body, 0)
    wait_scatter((NBLK - 1) % NBUF)
    plsc.subcore_barrier()

    row0 = sid * ROWS_PER_SUB
    pltpu.sync_copy(acc.at[pl.ds(row0, ROWS_PER_SUB)],
                    out_hbm.at[cid, pl.ds(row0, ROWS_PER_SUB)])

  return seg_sum(ms_lin, w, ridx)


def _tc_combine(partials):
  flat = partials.reshape(NC, N_RAYS * N_CH // 128, 128)
  rows = N_RAYS * N_CH // 128

  def add_body(p_ref, o_ref):
    o_ref[...] = p_ref[0] + p_ref[1]

  out = pl.pallas_call(
      add_body,
      out_shape=jax.ShapeDtypeStruct((rows, 128), jnp.float32),
      in_specs=[pl.BlockSpec((NC, rows, 128), lambda: (0, 0, 0))],
      out_specs=pl.BlockSpec((rows, 128), lambda: (0, 0)),
  )(flat)
  return out.reshape(N_RAYS, N_CH)


def kernel(ms, weights, ray_indices, num_rays):
  del num_rays
  ms_lin = (ms.reshape(N_SAMPLES // 128, 128, N_CH)
            .transpose(0, 2, 1)
            .reshape(N_SAMPLES // 128 * N_CH, 128))
  w = weights.reshape(N_SAMPLES)
  ridx = ray_indices.astype(jnp.int32)
  partials = _sc_segment_sum(ms_lin, w, ridx)
  return _tc_combine(partials)


# speedup vs baseline: 1.1545x; 1.1545x over previous
import functools

import jax
import jax.numpy as jnp
from jax import lax
from jax.experimental import pallas as pl
from jax.experimental.pallas import tpu as pltpu
from jax.experimental.pallas import tpu_sc as plsc

N_SAMPLES = 3145728
N_CH = 8
N_RAYS = 65536
NC = 2
NS = 16
NW = NC * NS
CHUNK = N_SAMPLES // NW
BLK = 2048
NBLK = CHUNK // BLK
GRP = BLK // 16
MS_ROWS = BLK // 128 * N_CH
ROWS_PER_SUB = N_RAYS // NS
NBUF = 2


def _sc_segment_sum(ms_lin, w, ridx):
  mesh = plsc.VectorSubcoreMesh(core_axis_name="c", subcore_axis_name="s")

  @functools.partial(
      pl.kernel,
      out_type=jax.ShapeDtypeStruct((NC, N_RAYS, N_CH), jnp.float32),
      mesh=mesh,
      scratch_types=dict(
          acc=pltpu.VMEM_SHARED((N_RAYS, N_CH), jnp.float32),
          ms_v=tuple(pltpu.VMEM((MS_ROWS, 128), jnp.float32)
                     for _ in range(NBUF)),
          w_v=tuple(pltpu.VMEM((BLK,), jnp.float32) for _ in range(NBUF)),
          idx_v=tuple(pltpu.VMEM((BLK,), jnp.int32) for _ in range(NBUF)),
          vals_v=tuple(pltpu.VMEM((BLK, N_CH), jnp.float32)
                       for _ in range(NBUF)),
          in_sems=tuple(pltpu.SemaphoreType.DMA((3,)) for _ in range(NBUF)),
          sc_sems=tuple(pltpu.SemaphoreType.DMA for _ in range(NBUF)),
      ),
      compiler_params=pltpu.CompilerParams(use_tc_tiling_on_sc=False,
                                           needs_layout_passes=False),
  )
  def seg_sum(ms_hbm, w_hbm, idx_hbm, out_hbm, *, acc, ms_v, w_v, idx_v,
              vals_v, in_sems, sc_sems):
    cid = lax.axis_index("c")
    sid = lax.axis_index("s")
    wid = cid * NS + sid

    iota = lax.iota(jnp.int32, 16)
    hi = iota >> 3
    lo = iota & 7
    zeros16 = jnp.zeros((16,), jnp.float32)

    def zero_body(i, _):
      plsc.store_scatter(vals_v[0], [2 * i + hi, lo], zeros16)
      return 0

    lax.fori_loop(0, BLK // 2, zero_body, 0)
    for rep in range(ROWS_PER_SUB // BLK):
      row0 = sid * ROWS_PER_SUB + rep * BLK
      pltpu.sync_copy(vals_v[0], acc.at[pl.ds(row0, BLK)])
    plsc.subcore_barrier()

    def start_in(b, k):
      s0 = pl.multiple_of(wid * CHUNK + b * BLK, BLK)
      r0 = pl.multiple_of(s0 // 16, MS_ROWS)
      pltpu.async_copy(ms_hbm.at[pl.ds(r0, MS_ROWS)], ms_v[k],
                       in_sems[k].at[0])
      pltpu.async_copy(w_hbm.at[pl.ds(s0, BLK)], w_v[k], in_sems[k].at[1])
      pltpu.async_copy(idx_hbm.at[pl.ds(s0, BLK)], idx_v[k],
                       in_sems[k].at[2])

    def wait_in(b, k):
      s0 = pl.multiple_of(wid * CHUNK + b * BLK, BLK)
      r0 = pl.multiple_of(s0 // 16, MS_ROWS)
      pltpu.make_async_copy(ms_hbm.at[pl.ds(r0, MS_ROWS)], ms_v[k],
                            in_sems[k].at[0]).wait()
      pltpu.make_async_copy(w_hbm.at[pl.ds(s0, BLK)], w_v[k],
                            in_sems[k].at[1]).wait()
      pltpu.make_async_copy(idx_hbm.at[pl.ds(s0, BLK)], idx_v[k],
                            in_sems[k].at[2]).wait()

    def wait_scatter(k):
      pltpu.make_async_copy(vals_v[k], acc.at[idx_v[k]], sc_sems[k]).wait()

    def compute_block(b, k):
      def store_group(samp, prods):
        for c in range(N_CH):
          plsc.store_scatter(vals_v[k],
                             [samp, jnp.full((16,), c, jnp.int32)],
                             prods[c])

      def group_body(g, carry):
        prev_samp, prev_prods = carry
        col = pl.multiple_of((g % (128 // 16)) * 16, 16)
        rowb = (g // (128 // 16)) * N_CH
        samp = 16 * g + iota
        w16 = w_v[k][pl.ds(pl.multiple_of(16 * g, 16), 16)]
        loads = [ms_v[k][rowb + c, pl.ds(col, 16)] for c in range(N_CH)]
        store_group(prev_samp, prev_prods)
        prods = tuple(m * w16 for m in loads)
        return samp, prods

      carry0 = (iota, tuple(zeros16 for _ in range(N_CH)))
      last = lax.fori_loop(0, GRP, group_body, carry0, unroll=2)
      store_group(*last)

    start_in(0, 0)

    def super_body(sb, _):
      for k in range(NBUF):
        b = sb * NBUF + k
        nk = (k + 1) % NBUF
        @pl.when(b >= 1)
        def _():
          wait_scatter(nk)

        @pl.when(b + 1 < NBLK)
        def _():
          start_in(b + 1, nk)

        wait_in(b, k)
        compute_block(b, k)
        pltpu.async_copy(vals_v[k], acc.at[idx_v[k]], sc_sems[k], add=True)
      return 0

    lax.fori_loop(0, NBLK // NBUF, super_body, 0)
    wait_scatter((NBLK - 1) % NBUF)
    plsc.subcore_barrier()

    row0 = sid * ROWS_PER_SUB
    pltpu.sync_copy(acc.at[pl.ds(row0, ROWS_PER_SUB)],
                    out_hbm.at[cid, pl.ds(row0, ROWS_PER_SUB)])

  return seg_sum(ms_lin, w, ridx)


def _sc_combine(partials):
  mesh = plsc.VectorSubcoreMesh(core_axis_name="c", subcore_axis_name="s")
  rays_per_w = N_RAYS // NW
  out_rows_per_w = rays_per_w // 128 * N_CH

  @functools.partial(
      pl.kernel,
      out_type=jax.ShapeDtypeStruct((N_RAYS // 128 * N_CH, 128),
                                    jnp.float32),
      mesh=mesh,
      scratch_types=dict(
          p0_v=pltpu.VMEM((rays_per_w, N_CH), jnp.float32),
          p1_v=pltpu.VMEM((rays_per_w, N_CH), jnp.float32),
          out_v=pltpu.VMEM((out_rows_per_w, 128), jnp.float32),
      ),
      compiler_params=pltpu.CompilerParams(use_tc_tiling_on_sc=False,
                                           needs_layout_passes=False),
  )
  def combine(p_hbm, out_hbm, *, p0_v, p1_v, out_v):
    cid = lax.axis_index("c")
    sid = lax.axis_index("s")
    wid = cid * NS + sid
    iota = lax.iota(jnp.int32, 16)

    ray0 = pl.multiple_of(wid * rays_per_w, rays_per_w)
    pltpu.sync_copy(p_hbm.at[0, pl.ds(ray0, rays_per_w)], p0_v)
    pltpu.sync_copy(p_hbm.at[1, pl.ds(ray0, rays_per_w)], p1_v)

    def body(i, _):
      j = i % 8
      c = (i // 8) % N_CH
      tt = i // (8 * N_CH)
      rows = 128 * tt + 16 * j + iota
      csplat = jnp.full((16,), 0, jnp.int32) + c
      a = plsc.load_gather(p0_v, [rows, csplat])
      b = plsc.load_gather(p1_v, [rows, csplat])
      out_v[tt * N_CH + c, pl.ds(pl.multiple_of(16 * j, 16), 16)] = a + b
      return 0

    lax.fori_loop(0, out_rows_per_w * 8, body, 0, unroll=2)

    orow0 = pl.multiple_of(wid * out_rows_per_w, out_rows_per_w)
    pltpu.sync_copy(out_v, out_hbm.at[pl.ds(orow0, out_rows_per_w)])

  return combine(partials)


def kernel(ms, weights, ray_indices, num_rays):
  del num_rays
  ms_lin = (ms.reshape(N_SAMPLES // 128, 128, N_CH)
            .transpose(0, 2, 1)
            .reshape(N_SAMPLES // 128 * N_CH, 128))
  w = weights.reshape(N_SAMPLES)
  ridx = ray_indices.astype(jnp.int32)
  partials = _sc_segment_sum(ms_lin, w, ridx)
  out_t = _sc_combine(partials)
  return (out_t.reshape(N_RAYS // 128, N_CH, 128)
          .transpose(0, 2, 1)
          .reshape(N_RAYS, N_CH))


# speedup vs baseline: 1.2802x; 1.1089x over previous
import functools

import jax
import jax.numpy as jnp
from jax import lax
from jax.experimental import pallas as pl
from jax.experimental.pallas import tpu as pltpu
from jax.experimental.pallas import tpu_sc as plsc

N_SAMPLES = 3145728
N_CH = 8
N_RAYS = 65536
NC = 2
NS = 16
NW = NC * NS
CHUNK = N_SAMPLES // NW
BLK = 2048
NBLK = CHUNK // BLK
GRP = BLK // 16
MS_ROWS = BLK // 128 * N_CH
ROWS_PER_SUB = N_RAYS // NS
NBUF = 2
NSBUF = 3
PERIOD = 6


def _sc_segment_sum(ms_lin, w, ridx):
  mesh = plsc.VectorSubcoreMesh(core_axis_name="c", subcore_axis_name="s")

  @functools.partial(
      pl.kernel,
      out_type=jax.ShapeDtypeStruct((NC, N_RAYS, N_CH), jnp.float32),
      mesh=mesh,
      scratch_types=dict(
          acc=pltpu.VMEM_SHARED((N_RAYS, N_CH), jnp.float32),
          ms_v=tuple(pltpu.VMEM((MS_ROWS, 128), jnp.float32)
                     for _ in range(NBUF)),
          w_v=tuple(pltpu.VMEM((BLK,), jnp.float32) for _ in range(NBUF)),
          idx_v=tuple(pltpu.VMEM((BLK,), jnp.int32) for _ in range(NSBUF)),
          vals_v=tuple(pltpu.VMEM((BLK, N_CH), jnp.float32)
                       for _ in range(NSBUF)),
          in_sems=tuple(pltpu.SemaphoreType.DMA((2,)) for _ in range(NBUF)),
          ix_sems=tuple(pltpu.SemaphoreType.DMA for _ in range(NSBUF)),
          sc_sems=tuple(pltpu.SemaphoreType.DMA for _ in range(NSBUF)),
      ),
      compiler_params=pltpu.CompilerParams(use_tc_tiling_on_sc=False,
                                           needs_layout_passes=False),
  )
  def seg_sum(ms_hbm, w_hbm, idx_hbm, out_hbm, *, acc, ms_v, w_v, idx_v,
              vals_v, in_sems, ix_sems, sc_sems):
    cid = lax.axis_index("c")
    sid = lax.axis_index("s")
    wid = cid * NS + sid

    iota = lax.iota(jnp.int32, 16)
    hi = iota >> 3
    lo = iota & 7
    zeros16 = jnp.zeros((16,), jnp.float32)

    def zero_body(i, _):
      plsc.store_scatter(vals_v[0], [2 * i + hi, lo], zeros16)
      return 0

    lax.fori_loop(0, BLK // 2, zero_body, 0)
    for rep in range(ROWS_PER_SUB // BLK):
      row0 = sid * ROWS_PER_SUB + rep * BLK
      pltpu.sync_copy(vals_v[0], acc.at[pl.ds(row0, BLK)])
    plsc.subcore_barrier()

    def start_in(b, ki, ks):
      s0 = pl.multiple_of(wid * CHUNK + b * BLK, BLK)
      r0 = pl.multiple_of(s0 // 16, MS_ROWS)
      pltpu.async_copy(ms_hbm.at[pl.ds(r0, MS_ROWS)], ms_v[ki],
                       in_sems[ki].at[0])
      pltpu.async_copy(w_hbm.at[pl.ds(s0, BLK)], w_v[ki], in_sems[ki].at[1])
      pltpu.async_copy(idx_hbm.at[pl.ds(s0, BLK)], idx_v[ks], ix_sems[ks])

    def wait_in(b, ki, ks):
      s0 = pl.multiple_of(wid * CHUNK + b * BLK, BLK)
      r0 = pl.multiple_of(s0 // 16, MS_ROWS)
      pltpu.make_async_copy(ms_hbm.at[pl.ds(r0, MS_ROWS)], ms_v[ki],
                            in_sems[ki].at[0]).wait()
      pltpu.make_async_copy(w_hbm.at[pl.ds(s0, BLK)], w_v[ki],
                            in_sems[ki].at[1]).wait()
      pltpu.make_async_copy(idx_hbm.at[pl.ds(s0, BLK)], idx_v[ks],
                            ix_sems[ks]).wait()

    def wait_scatter(ks):
      pltpu.make_async_copy(vals_v[ks], acc.at[idx_v[ks]],
                            sc_sems[ks]).wait()

    def compute_block(b, ki, ks):
      def store_group(samp, prods):
        for c in range(N_CH):
          plsc.store_scatter(vals_v[ks],
                             [samp, jnp.full((16,), c, jnp.int32)],
                             prods[c])

      def group_body(g, carry):
        prev_samp, prev_prods = carry
        col = pl.multiple_of((g % (128 // 16)) * 16, 16)
        rowb = (g // (128 // 16)) * N_CH
        samp = 16 * g + iota
        w16 = w_v[ki][pl.ds(pl.multiple_of(16 * g, 16), 16)]
        loads = [ms_v[ki][rowb + c, pl.ds(col, 16)] for c in range(N_CH)]
        store_group(prev_samp, prev_prods)
        prods = tuple(m * w16 for m in loads)
        return samp, prods

      carry0 = (iota, tuple(zeros16 for _ in range(N_CH)))
      last = lax.fori_loop(0, GRP, group_body, carry0, unroll=4)
      store_group(*last)

    start_in(0, 0, 0)

    def super_body(sb, _):
      for j in range(PERIOD):
        b = sb * PERIOD + j
        ki, ks = j % NBUF, j % NSBUF
        nki, nks = (j + 1) % NBUF, (j + 1) % NSBUF
        @pl.when(b >= 2)
        def _():
          wait_scatter(nks)

        @pl.when(b + 1 < NBLK)
        def _():
          start_in(b + 1, nki, nks)

        wait_in(b, ki, ks)
        compute_block(b, ki, ks)
        pltpu.async_copy(vals_v[ks], acc.at[idx_v[ks]], sc_sems[ks],
                         add=True)
      return 0

    lax.fori_loop(0, NBLK // PERIOD, super_body, 0)
    for b in (NBLK - 2, NBLK - 1):
      wait_scatter(b % NSBUF)
    plsc.subcore_barrier()

    row0 = sid * ROWS_PER_SUB
    pltpu.sync_copy(acc.at[pl.ds(row0, ROWS_PER_SUB)],
                    out_hbm.at[cid, pl.ds(row0, ROWS_PER_SUB)])

  return seg_sum(ms_lin, w, ridx)


def _sc_combine(partials):
  mesh = plsc.VectorSubcoreMesh(core_axis_name="c", subcore_axis_name="s")
  rays_per_w = N_RAYS // NW
  out_rows_per_w = rays_per_w // 128 * N_CH

  @functools.partial(
      pl.kernel,
      out_type=jax.ShapeDtypeStruct((N_RAYS // 128 * N_CH, 128),
                                    jnp.float32),
      mesh=mesh,
      scratch_types=dict(
          p0_v=pltpu.VMEM((rays_per_w, N_CH), jnp.float32),
          p1_v=pltpu.VMEM((rays_per_w, N_CH), jnp.float32),
          out_v=pltpu.VMEM((out_rows_per_w, 128), jnp.float32),
      ),
      compiler_params=pltpu.CompilerParams(use_tc_tiling_on_sc=False,
                                           needs_layout_passes=False),
  )
  def combine(p_hbm, out_hbm, *, p0_v, p1_v, out_v):
    cid = lax.axis_index("c")
    sid = lax.axis_index("s")
    wid = cid * NS + sid
    iota = lax.iota(jnp.int32, 16)

    ray0 = pl.multiple_of(wid * rays_per_w, rays_per_w)
    pltpu.sync_copy(p_hbm.at[0, pl.ds(ray0, rays_per_w)], p0_v)
    pltpu.sync_copy(p_hbm.at[1, pl.ds(ray0, rays_per_w)], p1_v)

    def body(i, _):
      j = i % 8
      c = (i // 8) % N_CH
      tt = i // (8 * N_CH)
      rows = 128 * tt + 16 * j + iota
      csplat = jnp.full((16,), 0, jnp.int32) + c
      a = plsc.load_gather(p0_v, [rows, csplat])
      b = plsc.load_gather(p1_v, [rows, csplat])
      out_v[tt * N_CH + c, pl.ds(pl.multiple_of(16 * j, 16), 16)] = a + b
      return 0

    lax.fori_loop(0, out_rows_per_w * 8, body, 0, unroll=2)

    orow0 = pl.multiple_of(wid * out_rows_per_w, out_rows_per_w)
    pltpu.sync_copy(out_v, out_hbm.at[pl.ds(orow0, out_rows_per_w)])

  return combine(partials)


def kernel(ms, weights, ray_indices, num_rays):
  del num_rays
  ms_lin = (ms.reshape(N_SAMPLES // 128, 128, N_CH)
            .transpose(0, 2, 1)
            .reshape(N_SAMPLES // 128 * N_CH, 128))
  w = weights.reshape(N_SAMPLES)
  ridx = ray_indices.astype(jnp.int32)
  partials = _sc_segment_sum(ms_lin, w, ridx)
  out_t = _sc_combine(partials)
  return (out_t.reshape(N_RAYS // 128, N_CH, 128)
          .transpose(0, 2, 1)
          .reshape(N_RAYS, N_CH))


# speedup vs baseline: 1.3010x; 1.0162x over previous
import functools

import jax
import jax.numpy as jnp
from jax import lax
from jax.experimental import pallas as pl
from jax.experimental.pallas import tpu as pltpu
from jax.experimental.pallas import tpu_sc as plsc

N_SAMPLES = 3145728
N_CH = 8
N_RAYS = 65536
NC = 2
NS = 16
NW = NC * NS
CHUNK = N_SAMPLES // NW
BLK = 2048
NBLK = CHUNK // BLK
GRP = BLK // 16
MS_ROWS = BLK // 128 * N_CH
ROWS_PER_SUB = N_RAYS // NS
NBUF = 2
NSBUF = 3
PERIOD = 6


def _sc_segment_sum(ms_lin, w, ridx):
  mesh = plsc.VectorSubcoreMesh(core_axis_name="c", subcore_axis_name="s")

  @functools.partial(
      pl.kernel,
      out_type=jax.ShapeDtypeStruct((NC, N_RAYS, N_CH), jnp.float32),
      mesh=mesh,
      scratch_types=dict(
          acc=pltpu.VMEM_SHARED((N_RAYS, N_CH), jnp.float32),
          ms_v=tuple(pltpu.VMEM((MS_ROWS, 128), jnp.float32)
                     for _ in range(NBUF)),
          w_v=tuple(pltpu.VMEM((BLK,), jnp.float32) for _ in range(NBUF)),
          idx_v=tuple(pltpu.VMEM((BLK,), jnp.int32) for _ in range(NSBUF)),
          vals_v=tuple(pltpu.VMEM((BLK, N_CH), jnp.float32)
                       for _ in range(NSBUF)),
          in_sems=tuple(pltpu.SemaphoreType.DMA((2,)) for _ in range(NBUF)),
          ix_sems=tuple(pltpu.SemaphoreType.DMA for _ in range(NSBUF)),
          sc_sems=tuple(pltpu.SemaphoreType.DMA for _ in range(NSBUF)),
      ),
      compiler_params=pltpu.CompilerParams(use_tc_tiling_on_sc=False,
                                           needs_layout_passes=False),
  )
  def seg_sum(ms_hbm, w_hbm, idx_hbm, out_hbm, *, acc, ms_v, w_v, idx_v,
              vals_v, in_sems, ix_sems, sc_sems):
    cid = lax.axis_index("c")
    sid = lax.axis_index("s")
    wid = cid * NS + sid

    iota = lax.iota(jnp.int32, 16)
    hi = iota >> 3
    lo = iota & 7
    zeros16 = jnp.zeros((16,), jnp.float32)

    def zero_body(i, _):
      plsc.store_scatter(vals_v[0], [2 * i + hi, lo], zeros16)
      return 0

    lax.fori_loop(0, BLK // 2, zero_body, 0)
    for rep in range(ROWS_PER_SUB // BLK):
      row0 = sid * ROWS_PER_SUB + rep * BLK
      pltpu.sync_copy(vals_v[0], acc.at[pl.ds(row0, BLK)])
    plsc.subcore_barrier()

    def start_in(b, ki, ks):
      s0 = pl.multiple_of(wid * CHUNK + b * BLK, BLK)
      r0 = pl.multiple_of(s0 // 16, MS_ROWS)
      pltpu.async_copy(ms_hbm.at[pl.ds(r0, MS_ROWS)], ms_v[ki],
                       in_sems[ki].at[0])
      pltpu.async_copy(w_hbm.at[pl.ds(s0, BLK)], w_v[ki], in_sems[ki].at[1])
      pltpu.async_copy(idx_hbm.at[pl.ds(s0, BLK)], idx_v[ks], ix_sems[ks])

    def wait_in(b, ki, ks):
      s0 = pl.multiple_of(wid * CHUNK + b * BLK, BLK)
      r0 = pl.multiple_of(s0 // 16, MS_ROWS)
      pltpu.make_async_copy(ms_hbm.at[pl.ds(r0, MS_ROWS)], ms_v[ki],
                            in_sems[ki].at[0]).wait()
      pltpu.make_async_copy(w_hbm.at[pl.ds(s0, BLK)], w_v[ki],
                            in_sems[ki].at[1]).wait()
      pltpu.make_async_copy(idx_hbm.at[pl.ds(s0, BLK)], idx_v[ks],
                            ix_sems[ks]).wait()

    def wait_scatter(ks):
      pltpu.make_async_copy(vals_v[ks], acc.at[idx_v[ks]],
                            sc_sems[ks]).wait()

    def compute_block(b, ki, ks):
      def store_group(samp, prods):
        for c in range(N_CH):
          plsc.store_scatter(vals_v[ks],
                             [samp, jnp.full((16,), c, jnp.int32)],
                             prods[c])

      def group_body(g, carry):
        prev_samp, prev_prods = carry
        col = pl.multiple_of((g % (128 // 16)) * 16, 16)
        rowb = (g // (128 // 16)) * N_CH
        samp = 16 * g + iota
        w16 = w_v[ki][pl.ds(pl.multiple_of(16 * g, 16), 16)]
        loads = [ms_v[ki][rowb + c, pl.ds(col, 16)] for c in range(N_CH)]
        store_group(prev_samp, prev_prods)
        prods = tuple(m * w16 for m in loads)
        return samp, prods

      carry0 = (iota, tuple(zeros16 for _ in range(N_CH)))
      last = lax.fori_loop(0, GRP, group_body, carry0, unroll=8)
      store_group(*last)

    start_in(0, 0, 0)

    def super_body(sb, _):
      for j in range(PERIOD):
        b = sb * PERIOD + j
        ki, ks = j % NBUF, j % NSBUF
        nki, nks = (j + 1) % NBUF, (j + 1) % NSBUF
        @pl.when(b >= NSBUF - 1)
        def _():
          wait_scatter(nks)

        @pl.when(b + 1 < NBLK)
        def _():
          start_in(b + 1, nki, nks)

        wait_in(b, ki, ks)
        compute_block(b, ki, ks)
        pltpu.async_copy(vals_v[ks], acc.at[idx_v[ks]], sc_sems[ks],
                         add=True)
      return 0

    lax.fori_loop(0, NBLK // PERIOD, super_body, 0)
    for b in range(NBLK - NSBUF + 1, NBLK):
      wait_scatter(b % NSBUF)
    plsc.subcore_barrier()

    row0 = sid * ROWS_PER_SUB
    pltpu.sync_copy(acc.at[pl.ds(row0, ROWS_PER_SUB)],
                    out_hbm.at[cid, pl.ds(row0, ROWS_PER_SUB)])

  return seg_sum(ms_lin, w, ridx)


def _sc_combine(partials):
  mesh = plsc.VectorSubcoreMesh(core_axis_name="c", subcore_axis_name="s")
  rays_per_w = N_RAYS // NW
  out_rows_per_w = rays_per_w // 128 * N_CH

  @functools.partial(
      pl.kernel,
      out_type=jax.ShapeDtypeStruct((N_RAYS // 128 * N_CH, 128),
                                    jnp.float32),
      mesh=mesh,
      scratch_types=dict(
          p0_v=pltpu.VMEM((rays_per_w, N_CH), jnp.float32),
          p1_v=pltpu.VMEM((rays_per_w, N_CH), jnp.float32),
          out_v=pltpu.VMEM((out_rows_per_w, 128), jnp.float32),
      ),
      compiler_params=pltpu.CompilerParams(use_tc_tiling_on_sc=False,
                                           needs_layout_passes=False),
  )
  def combine(p_hbm, out_hbm, *, p0_v, p1_v, out_v):
    cid = lax.axis_index("c")
    sid = lax.axis_index("s")
    wid = cid * NS + sid
    iota = lax.iota(jnp.int32, 16)

    ray0 = pl.multiple_of(wid * rays_per_w, rays_per_w)
    pltpu.sync_copy(p_hbm.at[0, pl.ds(ray0, rays_per_w)], p0_v)
    pltpu.sync_copy(p_hbm.at[1, pl.ds(ray0, rays_per_w)], p1_v)

    def body(i, _):
      j = i % 8
      c = (i // 8) % N_CH
      tt = i // (8 * N_CH)
      rows = 128 * tt + 16 * j + iota
      csplat = jnp.full((16,), 0, jnp.int32) + c
      a = plsc.load_gather(p0_v, [rows, csplat])
      b = plsc.load_gather(p1_v, [rows, csplat])
      out_v[tt * N_CH + c, pl.ds(pl.multiple_of(16 * j, 16), 16)] = a + b
      return 0

    lax.fori_loop(0, out_rows_per_w * 8, body, 0, unroll=2)

    orow0 = pl.multiple_of(wid * out_rows_per_w, out_rows_per_w)
    pltpu.sync_copy(out_v, out_hbm.at[pl.ds(orow0, out_rows_per_w)])

  return combine(partials)


def kernel(ms, weights, ray_indices, num_rays):
  del num_rays
  ms_lin = (ms.reshape(N_SAMPLES // 128, 128, N_CH)
            .transpose(0, 2, 1)
            .reshape(N_SAMPLES // 128 * N_CH, 128))
  w = weights.reshape(N_SAMPLES)
  ridx = ray_indices.astype(jnp.int32)
  partials = _sc_segment_sum(ms_lin, w, ridx)
  out_t = _sc_combine(partials)
  return (out_t.reshape(N_RAYS // 128, N_CH, 128)
          .transpose(0, 2, 1)
          .reshape(N_RAYS, N_CH))


# speedup vs baseline: 1.3289x; 1.0214x over previous
import functools

import jax
import jax.numpy as jnp
from jax import lax
from jax.experimental import pallas as pl
from jax.experimental.pallas import tpu as pltpu
from jax.experimental.pallas import tpu_sc as plsc

N_SAMPLES = 3145728
N_CH = 8
N_RAYS = 65536
NC = 2
NS = 16
NW = NC * NS
CHUNK = N_SAMPLES // NW
BLK = 2048
NBLK = CHUNK // BLK
GRP = BLK // 16
MS_ROWS = BLK // 128 * N_CH
ROWS_PER_SUB = N_RAYS // NS
NBUF = 2
NSBUF = 3
PERIOD = 6


def _sc_segment_sum(ms_lin, w, ridx):
  mesh = plsc.VectorSubcoreMesh(core_axis_name="c", subcore_axis_name="s")

  @functools.partial(
      pl.kernel,
      out_type=jax.ShapeDtypeStruct((NC, N_RAYS, N_CH), jnp.float32),
      mesh=mesh,
      scratch_types=dict(
          acc=pltpu.VMEM_SHARED((N_RAYS, N_CH), jnp.float32),
          ms_v=tuple(pltpu.VMEM((MS_ROWS, 128), jnp.float32)
                     for _ in range(NBUF)),
          w_v=tuple(pltpu.VMEM((BLK,), jnp.float32) for _ in range(NBUF)),
          idx_v=tuple(pltpu.VMEM((BLK,), jnp.int32) for _ in range(NSBUF)),
          vals_v=tuple(pltpu.VMEM((BLK, N_CH), jnp.float32)
                       for _ in range(NSBUF)),
          in_sems=tuple(pltpu.SemaphoreType.DMA((2,)) for _ in range(NBUF)),
          ix_sems=tuple(pltpu.SemaphoreType.DMA for _ in range(NSBUF)),
          sc_sems=tuple(pltpu.SemaphoreType.DMA for _ in range(NSBUF)),
      ),
      compiler_params=pltpu.CompilerParams(use_tc_tiling_on_sc=False,
                                           needs_layout_passes=False),
  )
  def seg_sum(ms_hbm, w_hbm, idx_hbm, out_hbm, *, acc, ms_v, w_v, idx_v,
              vals_v, in_sems, ix_sems, sc_sems):
    cid = lax.axis_index("c")
    sid = lax.axis_index("s")
    wid = cid * NS + sid

    iota = lax.iota(jnp.int32, 16)
    hi = iota >> 3
    lo = iota & 7
    zeros16 = jnp.zeros((16,), jnp.float32)

    def zero_body(i, _):
      plsc.store_scatter(vals_v[0], [2 * i + hi, lo], zeros16)
      return 0

    lax.fori_loop(0, BLK // 2, zero_body, 0)
    for rep in range(ROWS_PER_SUB // BLK):
      row0 = sid * ROWS_PER_SUB + rep * BLK
      pltpu.sync_copy(vals_v[0], acc.at[pl.ds(row0, BLK)])
    plsc.subcore_barrier()

    def start_in(b, ki, ks):
      s0 = pl.multiple_of(wid * CHUNK + b * BLK, BLK)
      r0 = pl.multiple_of(s0 // 16, MS_ROWS)
      pltpu.async_copy(ms_hbm.at[pl.ds(r0, MS_ROWS)], ms_v[ki],
                       in_sems[ki].at[0])
      pltpu.async_copy(w_hbm.at[pl.ds(s0, BLK)], w_v[ki], in_sems[ki].at[1])
      pltpu.async_copy(idx_hbm.at[pl.ds(s0, BLK)], idx_v[ks], ix_sems[ks])

    def wait_in(b, ki, ks):
      s0 = pl.multiple_of(wid * CHUNK + b * BLK, BLK)
      r0 = pl.multiple_of(s0 // 16, MS_ROWS)
      pltpu.make_async_copy(ms_hbm.at[pl.ds(r0, MS_ROWS)], ms_v[ki],
                            in_sems[ki].at[0]).wait()
      pltpu.make_async_copy(w_hbm.at[pl.ds(s0, BLK)], w_v[ki],
                            in_sems[ki].at[1]).wait()
      pltpu.make_async_copy(idx_hbm.at[pl.ds(s0, BLK)], idx_v[ks],
                            ix_sems[ks]).wait()

    def wait_scatter(ks):
      pltpu.make_async_copy(vals_v[ks], acc.at[idx_v[ks]],
                            sc_sems[ks]).wait()

    def compute_block(b, ki, ks):
      def store_group(samp, prods):
        for c in range(N_CH):
          plsc.store_scatter(vals_v[ks],
                             [samp, jnp.full((16,), c, jnp.int32)],
                             prods[c])

      def group_body(g, carry):
        prev_samp, prev_prods = carry
        col = pl.multiple_of((g % (128 // 16)) * 16, 16)
        rowb = (g // (128 // 16)) * N_CH
        samp = 16 * g + iota
        w16 = w_v[ki][pl.ds(pl.multiple_of(16 * g, 16), 16)]
        loads = [ms_v[ki][rowb + c, pl.ds(col, 16)] for c in range(N_CH)]
        store_group(prev_samp, prev_prods)
        prods = tuple(m * w16 for m in loads)
        return samp, prods

      carry0 = (iota, tuple(zeros16 for _ in range(N_CH)))
      last = lax.fori_loop(0, GRP, group_body, carry0, unroll=8)
      store_group(*last)

    start_in(0, 0, 0)

    def super_body(sb, _):
      for j in range(PERIOD):
        b = sb * PERIOD + j
        ki, ks = j % NBUF, j % NSBUF
        nki, nks = (j + 1) % NBUF, (j + 1) % NSBUF
        @pl.when(b >= NSBUF - 1)
        def _():
          wait_scatter(nks)

        @pl.when(b + 1 < NBLK)
        def _():
          start_in(b + 1, nki, nks)

        wait_in(b, ki, ks)
        compute_block(b, ki, ks)
        pltpu.async_copy(vals_v[ks], acc.at[idx_v[ks]], sc_sems[ks],
                         add=True)
      return 0

    lax.fori_loop(0, NBLK // PERIOD, super_body, 0)
    for b in range(NBLK - NSBUF + 1, NBLK):
      wait_scatter(b % NSBUF)
    plsc.subcore_barrier()

    row0 = sid * ROWS_PER_SUB
    pltpu.sync_copy(acc.at[pl.ds(row0, ROWS_PER_SUB)],
                    out_hbm.at[cid, pl.ds(row0, ROWS_PER_SUB)])

  return seg_sum(ms_lin, w, ridx)


def _sc_combine(partials):
  mesh = plsc.VectorSubcoreMesh(core_axis_name="c", subcore_axis_name="s")
  rays_per_w = N_RAYS // NW
  out_rows_per_w = rays_per_w // 128 * N_CH

  @functools.partial(
      pl.kernel,
      out_type=jax.ShapeDtypeStruct((N_RAYS // 128 * N_CH, 128),
                                    jnp.float32),
      mesh=mesh,
      scratch_types=dict(
          p0_v=pltpu.VMEM((rays_per_w, N_CH), jnp.float32),
          p1_v=pltpu.VMEM((rays_per_w, N_CH), jnp.float32),
          out_v=pltpu.VMEM((out_rows_per_w, 128), jnp.float32),
      ),
      compiler_params=pltpu.CompilerParams(use_tc_tiling_on_sc=False,
                                           needs_layout_passes=False),
  )
  def combine(p_hbm, out_hbm, *, p0_v, p1_v, out_v):
    cid = lax.axis_index("c")
    sid = lax.axis_index("s")
    wid = cid * NS + sid
    iota = lax.iota(jnp.int32, 16)

    ray0 = pl.multiple_of(wid * rays_per_w, rays_per_w)
    pltpu.sync_copy(p_hbm.at[0, pl.ds(ray0, rays_per_w)], p0_v)
    pltpu.sync_copy(p_hbm.at[1, pl.ds(ray0, rays_per_w)], p1_v)

    def body(r, _):
      c = r % N_CH
      tt = r // N_CH
      csplat = jnp.full((16,), 0, jnp.int32) + c
      rows = [128 * tt + 16 * j + iota for j in range(8)]
      a = [plsc.load_gather(p0_v, [rw, csplat]) for rw in rows]
      b = [plsc.load_gather(p1_v, [rw, csplat]) for rw in rows]
      for j in range(8):
        out_v[r, pl.ds(pl.multiple_of(16 * j, 16), 16)] = a[j] + b[j]
      return 0

    lax.fori_loop(0, out_rows_per_w, body, 0, unroll=2)

    orow0 = pl.multiple_of(wid * out_rows_per_w, out_rows_per_w)
    pltpu.sync_copy(out_v, out_hbm.at[pl.ds(orow0, out_rows_per_w)])

  return combine(partials)


def kernel(ms, weights, ray_indices, num_rays):
  del num_rays
  ms_lin = (ms.reshape(N_SAMPLES // 128, 128, N_CH)
            .transpose(0, 2, 1)
            .reshape(N_SAMPLES // 128 * N_CH, 128))
  w = weights.reshape(N_SAMPLES)
  ridx = ray_indices.astype(jnp.int32)
  partials = _sc_segment_sum(ms_lin, w, ridx)
  out_t = _sc_combine(partials)
  return (out_t.reshape(N_RAYS // 128, N_CH, 128)
          .transpose(0, 2, 1)
          .reshape(N_RAYS, N_CH))
